# baseline (speedup 1.0000x reference)
import functools

import jax
import jax.numpy as jnp
from jax import lax
from jax.experimental import pallas as pl
from jax.experimental.pallas import tpu as pltpu
from jax.experimental.pallas import tpu_sc as plsc

NUM_EMB = 1_000_000
DIM = 32
BATCH = 16384
HIST = 50
TOTAL = BATCH * HIST

NC, NS = 2, 16
NW = NC * NS
PER_W = TOTAL // NW
CHUNK = 128
SUPER = 512
N_SUB = SUPER // CHUNK
N_OUTER = PER_W // SUPER
IDX_ROWS = PER_W // CHUNK


def _body(idx_hbm, wr_hbm, wi_hbm, outr_hbm, outi_hbm,
          idx_v, rows_r, rows_i, sem_r, sem_i):
    wid = lax.axis_index("s") * NC + lax.axis_index("c")
    pltpu.sync_copy(idx_hbm.at[pl.ds(wid * IDX_ROWS, IDX_ROWS)], idx_v)
    base = wid * PER_W

    @pl.loop(0, N_OUTER)
    def _outer(c):
        copies = []
        for j in range(N_SUB):
            row = c * N_SUB + j
            dst = pl.ds(j * CHUNK, CHUNK)
            copies.append(pltpu.async_copy(
                wr_hbm.at[idx_v.at[row]], rows_r.at[dst], sem_r))
            copies.append(pltpu.async_copy(
                wi_hbm.at[idx_v.at[row]], rows_i.at[dst], sem_i))
        for cp in copies:
            cp.wait()
        out = pl.ds(base + c * SUPER, SUPER)
        pltpu.sync_copy(rows_r, outr_hbm.at[out])
        pltpu.sync_copy(rows_i, outi_hbm.at[out])


@jax.jit
def _run(idx2d, w_real, w_imag):
    mesh = plsc.VectorSubcoreMesh(
        core_axis_name="c", subcore_axis_name="s",
        num_cores=NC, num_subcores=NS)
    f = pl.kernel(
        _body,
        out_type=[
            jax.ShapeDtypeStruct((TOTAL, DIM), jnp.float32),
            jax.ShapeDtypeStruct((TOTAL, DIM), jnp.float32),
        ],
        mesh=mesh,
        scratch_types=[
            pltpu.VMEM((IDX_ROWS, CHUNK), jnp.int32),
            pltpu.VMEM((SUPER, DIM), jnp.float32),
            pltpu.VMEM((SUPER, DIM), jnp.float32),
            pltpu.SemaphoreType.DMA,
            pltpu.SemaphoreType.DMA,
        ],
        compiler_params=pltpu.CompilerParams(use_tc_tiling_on_sc=False),
    )
    return f(idx2d, w_real, w_imag)


def kernel(input, W_real, W_imag):
    idx2d = input.reshape(TOTAL // CHUNK, CHUNK)
    r, i = _run(idx2d, W_real, W_imag)
    c = lax.complex(r.reshape(-1), i.reshape(-1))
    c = lax.optimization_barrier(c)
    return c.reshape(BATCH, HIST, DIM)


# speedup vs baseline: 2.2917x; 2.2917x over previous
import jax
import jax.numpy as jnp
from jax import lax
from jax.experimental import pallas as pl
from jax.experimental.pallas import tpu as pltpu
from jax.experimental.pallas import tpu_sc as plsc

NUM_EMB = 1_000_000
DIM = 32
BATCH = 16384
HIST = 50

NC, NS = 2, 16
NW = NC * NS
WB = BATCH // NW
N_STREAM = WB // 128
BB = WB // 16


def _body(idx_hbm, wr_hbm, wi_hbm, outr_hbm, outi_hbm,
          idx_v, rows_r, rows_i, out_vr, out_vi, sem_g):
    wid = lax.axis_index("s") * NC + lax.axis_index("c")
    wb0 = wid * WB
    pltpu.sync_copy(idx_hbm.at[:, pl.ds(wb0, WB)], idx_v)
    iota = lax.iota(jnp.int32, 16)

    @pl.loop(0, HIST)
    def _per_h(h):
        gathers = []
        for s in range(N_STREAM):
            rng = pl.ds(s * 128, 128)
            gathers.append(pltpu.async_copy(
                wr_hbm.at[idx_v.at[h, rng]], rows_r.at[rng], sem_g))
            gathers.append(pltpu.async_copy(
                wi_hbm.at[idx_v.at[h, rng]], rows_i.at[rng], sem_g))
        for cp in gathers:
            cp.wait()

        for rows_v, out_v in ((rows_r, out_vr), (rows_i, out_vi)):
            @pl.loop(0, BB)
            def _per_bb(bb):
                bvec = bb * 16 + iota
                for d in range(DIM):
                    out_v[d, pl.ds(bb * 16, 16)] = plsc.load_gather(
                        rows_v, [bvec, jnp.full((16,), d, jnp.int32)])

        pltpu.sync_copy(
            out_vr, outr_hbm.at[pl.ds(h * DIM, DIM), pl.ds(wb0, WB)])
        pltpu.sync_copy(
            out_vi, outi_hbm.at[pl.ds(h * DIM, DIM), pl.ds(wb0, WB)])


@jax.jit
def _run(idx2d, w_real, w_imag):
    mesh = plsc.VectorSubcoreMesh(
        core_axis_name="c", subcore_axis_name="s",
        num_cores=NC, num_subcores=NS)
    f = pl.kernel(
        _body,
        out_type=[
            jax.ShapeDtypeStruct((HIST * DIM, BATCH), jnp.float32),
            jax.ShapeDtypeStruct((HIST * DIM, BATCH), jnp.float32),
        ],
        mesh=mesh,
        scratch_types=[
            pltpu.VMEM((HIST, WB), jnp.int32),
            pltpu.VMEM((WB, DIM), jnp.float32),
            pltpu.VMEM((WB, DIM), jnp.float32),
            pltpu.VMEM((DIM, WB), jnp.float32),
            pltpu.VMEM((DIM, WB), jnp.float32),
            pltpu.SemaphoreType.DMA,
        ],
        compiler_params=pltpu.CompilerParams(
            use_tc_tiling_on_sc=False, needs_layout_passes=False),
    )
    return f(idx2d, w_real, w_imag)


def kernel(input, W_real, W_imag):
    idx2d = input.T
    zr, zi = _run(idx2d, W_real, W_imag)
    r = zr.reshape(HIST, DIM, BATCH)
    i = zi.reshape(HIST, DIM, BATCH)
    c = lax.complex(r, i)
    return jnp.transpose(c, (2, 0, 1))
